# baseline (speedup 1.0000x reference)
import functools

import jax
import jax.numpy as jnp
from jax import lax
from jax.experimental import pallas as pl
from jax.experimental.pallas import tpu as pltpu
from jax.experimental.pallas import tpu_sc as plsc

_INFO = plsc.get_sparse_core_info()
_NC = _INFO.num_cores
_NS = _INFO.num_subcores
_NW = _NC * _NS

_GRP = 16
_T = 4096


def _num_lines(vocab):
    main = (vocab // (2 * _T)) * 2 * _T
    return main // 2 + (vocab - main)


@functools.lru_cache(maxsize=None)
def _make_transpose(vocab, dim):
    nblk = vocab // (2 * _T)

    def body(l_ref, r_ref, o_ref):
        o_ref[:, 0:dim] = l_ref[...].T
        o_ref[:, dim:2 * dim] = r_ref[...].T

    return pl.pallas_call(
        body,
        grid=(nblk,),
        in_specs=[
            pl.BlockSpec((dim, _T), lambda g: (0, 2 * g)),
            pl.BlockSpec((dim, _T), lambda g: (0, 2 * g + 1)),
        ],
        out_specs=pl.BlockSpec((_T, 2 * dim), lambda g: (g, 0)),
        out_shape=jax.ShapeDtypeStruct(
            (_num_lines(vocab), 2 * dim), jnp.float32
        ),
    )


@functools.lru_cache(maxsize=None)
def _make_gather(batch, vocab, dim):
    assert batch % (8 * _NW) == 0 and dim == 64
    b_per_w = batch // _NW
    mesh = plsc.VectorSubcoreMesh(core_axis_name="c", subcore_axis_name="s")

    @functools.partial(
        pl.kernel,
        mesh=mesh,
        out_type=jax.ShapeDtypeStruct((batch, dim), jnp.float32),
        scratch_types=[
            pltpu.VMEM((b_per_w,), jnp.int32),
            pltpu.VMEM((b_per_w,), jnp.int32),
            pltpu.VMEM((b_per_w // 2, 2 * dim), jnp.float32),
            pltpu.VMEM((b_per_w, dim), jnp.float32),
            pltpu.SemaphoreType.DMA,
        ],
    )
    def gather(table_hbm, idx_hbm, out_hbm, idx_v, off_v, pairs_v, rows_v,
               sem):
        wid = lax.axis_index("s") * _NC + lax.axis_index("c")
        base = wid * b_per_w
        half_b = b_per_w // 2
        pltpu.sync_copy(idx_hbm.at[pl.ds(base, b_per_w)], idx_v)

        def prep(k, _):
            r = idx_v[pl.ds(k * 16, 16)]
            idx_v[pl.ds(k * 16, 16)] = (
                ((r >> 13) << 12) + (r & (_T - 1))
            )
            off_v[pl.ds(k * 16, 16)] = ((r >> 12) & 1) << 6
            return ()

        lax.fori_loop(0, b_per_w // 16, prep, (), unroll=True)

        for h in range(2):
            h0 = h * half_b

            def group(g, _):
                i0 = h0 + g * _GRP
                s0 = g * _GRP
                line_vec = idx_v[pl.ds(i0, _GRP)]
                copies = []
                for j in range(_GRP):
                    copies.append(
                        pltpu.async_copy(
                            table_hbm.at[pl.ds(line_vec[j], 1), :],
                            pairs_v.at[pl.ds(s0 + j, 1), :],
                            sem,
                        )
                    )
                for c in copies:
                    c.wait()
                return ()

            lax.fori_loop(0, half_b // _GRP, group, (), unroll=False)

            def extract(s, _):
                i = h0 + s
                off = off_v[pl.ds(i, 1)][0]
                for k in range(4):
                    rows_v[i, pl.ds(k * 16, 16)] = (
                        pairs_v[s, pl.ds(off + k * 16, 16)]
                    )
                return ()

            lax.fori_loop(0, half_b, extract, (), unroll=False)

        pltpu.sync_copy(rows_v, out_hbm.at[pl.ds(base, b_per_w)])

    return gather


@jax.jit
def kernel(codes, emb_weight):
    batch, = codes.shape
    vocab, dim = emb_weight.shape
    main = (vocab // (2 * _T)) * 2 * _T
    table_pairs = _make_transpose(vocab, dim)(emb_weight.T, emb_weight.T)
    if main != vocab:
        table_pairs = lax.dynamic_update_slice(
            table_pairs, emb_weight[main:, :], (main // 2, 0)
        )
    gather = _make_gather(batch, vocab, dim)
    return gather(table_pairs, codes.astype(jnp.int32))


# speedup vs baseline: 1.0784x; 1.0784x over previous
import functools

import jax
import jax.numpy as jnp
from jax import lax
from jax.experimental import pallas as pl
from jax.experimental.pallas import tpu as pltpu
from jax.experimental.pallas import tpu_sc as plsc

_INFO = plsc.get_sparse_core_info()
_NC = _INFO.num_cores
_NS = _INFO.num_subcores
_NW = _NC * _NS

_GRP = 16
_TCHUNK = 15872


@functools.lru_cache(maxsize=None)
def _make_transpose(vocab, dim):
    main = (vocab // _TCHUNK) * _TCHUNK

    def body(x_ref, o_ref):
        row = lax.broadcasted_iota(jnp.int32, (dim, dim), 0)
        col = lax.broadcasted_iota(jnp.int32, (dim, dim), 1)
        eye = (row == col).astype(jnp.float32)
        o_ref[...] = lax.dot_general(
            x_ref[...], eye,
            ((( 0,), (0,)), ((), ())),
            preferred_element_type=jnp.float32,
        )

    return pl.pallas_call(
        body,
        grid=(main // _TCHUNK,),
        in_specs=[
            pl.BlockSpec((dim, _TCHUNK), lambda g: (0, g)),
        ],
        out_specs=pl.BlockSpec((_TCHUNK, dim), lambda g: (g, 0)),
        out_shape=jax.ShapeDtypeStruct((vocab, dim), jnp.float32),
    )


@functools.lru_cache(maxsize=None)
def _make_gather(batch, vocab, dim):
    assert batch % (8 * _NW) == 0 and dim == 64
    b_per_w = batch // _NW
    mesh = plsc.VectorSubcoreMesh(core_axis_name="c", subcore_axis_name="s")

    @functools.partial(
        pl.kernel,
        mesh=mesh,
        out_type=jax.ShapeDtypeStruct((batch, dim), jnp.float32),
        scratch_types=[
            pltpu.VMEM((b_per_w,), jnp.int32),
            pltpu.VMEM((b_per_w, dim), jnp.float32),
            pltpu.SemaphoreType.DMA,
        ],
    )
    def gather(table_hbm, idx_hbm, out_hbm, idx_v, rows_v, sem):
        wid = lax.axis_index("s") * _NC + lax.axis_index("c")
        base = wid * b_per_w
        pltpu.sync_copy(idx_hbm.at[pl.ds(base, b_per_w)], idx_v)

        def group(g, _):
            i0 = g * _GRP
            idx_vec = idx_v[pl.ds(i0, _GRP)]
            copies = []
            for j in range(_GRP):
                r = idx_vec[j]
                copies.append(
                    pltpu.async_copy(
                        table_hbm.at[pl.ds(r, 1), :],
                        rows_v.at[pl.ds(i0 + j, 1), :],
                        sem,
                    )
                )
            for c in copies:
                c.wait()
            return ()

        lax.fori_loop(0, b_per_w // _GRP, group, (), unroll=False)
        pltpu.sync_copy(rows_v, out_hbm.at[pl.ds(base, b_per_w)])

    return gather


@jax.jit
def kernel(codes, emb_weight):
    batch, = codes.shape
    vocab, dim = emb_weight.shape
    main = (vocab // _TCHUNK) * _TCHUNK
    table_rm = _make_transpose(vocab, dim)(emb_weight.T)
    if main != vocab:
        table_rm = lax.dynamic_update_slice(
            table_rm, emb_weight[main:, :], (main, 0)
        )
    gather = _make_gather(batch, vocab, dim)
    return gather(table_rm, codes.astype(jnp.int32))


# speedup vs baseline: 1.1390x; 1.0562x over previous
import functools

import jax
import jax.numpy as jnp
from jax import lax
from jax.experimental import pallas as pl
from jax.experimental.pallas import tpu as pltpu
from jax.experimental.pallas import tpu_sc as plsc

_INFO = plsc.get_sparse_core_info()
_NC = _INFO.num_cores
_NS = _INFO.num_subcores
_NW = _NC * _NS

_GRP = 32
_TCHUNK = 27776


@functools.lru_cache(maxsize=None)
def _make_transpose(vocab, dim):
    main = (vocab // _TCHUNK) * _TCHUNK

    def body(x_ref, o_ref):
        o_ref[...] = x_ref[...].T

    return pl.pallas_call(
        body,
        grid=(main // _TCHUNK,),
        in_specs=[
            pl.BlockSpec((dim, _TCHUNK), lambda g: (0, g)),
        ],
        out_specs=pl.BlockSpec((_TCHUNK, dim), lambda g: (g, 0)),
        out_shape=jax.ShapeDtypeStruct((vocab, dim), jnp.float32),
    )


@functools.lru_cache(maxsize=None)
def _make_gather(batch, vocab, dim):
    assert batch % (8 * _NW) == 0 and dim == 64
    b_per_w = batch // _NW
    mesh = plsc.VectorSubcoreMesh(core_axis_name="c", subcore_axis_name="s")

    @functools.partial(
        pl.kernel,
        mesh=mesh,
        out_type=jax.ShapeDtypeStruct((batch, dim), jnp.float32),
        scratch_types=[
            pltpu.VMEM((b_per_w,), jnp.int32),
            pltpu.VMEM((b_per_w, dim), jnp.float32),
            pltpu.SemaphoreType.DMA,
        ],
    )
    def gather(table_hbm, idx_hbm, out_hbm, idx_v, rows_v, sem):
        wid = lax.axis_index("s") * _NC + lax.axis_index("c")
        base = wid * b_per_w
        pltpu.sync_copy(idx_hbm.at[pl.ds(base, b_per_w)], idx_v)

        def group(g, _):
            i0 = g * _GRP
            idx_vec = idx_v[pl.ds(i0, _GRP)]
            copies = []
            for j in range(_GRP):
                r = idx_vec[j]
                copies.append(
                    pltpu.async_copy(
                        table_hbm.at[pl.ds(r, 1), :],
                        rows_v.at[pl.ds(i0 + j, 1), :],
                        sem,
                    )
                )
            for c in copies:
                c.wait()
            return ()

        lax.fori_loop(0, b_per_w // _GRP, group, (), unroll=False)
        pltpu.sync_copy(rows_v, out_hbm.at[pl.ds(base, b_per_w)])

    return gather


@jax.jit
def kernel(codes, emb_weight):
    batch, = codes.shape
    vocab, dim = emb_weight.shape
    main = (vocab // _TCHUNK) * _TCHUNK
    table_rm = _make_transpose(vocab, dim)(emb_weight.T)
    if main != vocab:
        table_rm = lax.dynamic_update_slice(
            table_rm, emb_weight[main:, :], (main, 0)
        )
    gather = _make_gather(batch, vocab, dim)
    return gather(table_rm, codes.astype(jnp.int32))


# speedup vs baseline: 1.1459x; 1.0060x over previous
import functools

import jax
import jax.numpy as jnp
from jax import lax
from jax.experimental import pallas as pl
from jax.experimental.pallas import tpu as pltpu
from jax.experimental.pallas import tpu_sc as plsc

_INFO = plsc.get_sparse_core_info()
_NC = _INFO.num_cores
_NS = _INFO.num_subcores
_NW = _NC * _NS

_GRP = 32
_TCHUNK = 35712


@functools.lru_cache(maxsize=None)
def _make_transpose(vocab, dim):
    main = (vocab // _TCHUNK) * _TCHUNK

    def body(x_ref, o_ref):
        o_ref[...] = x_ref[...].T

    return pl.pallas_call(
        body,
        grid=(main // _TCHUNK,),
        in_specs=[
            pl.BlockSpec((dim, _TCHUNK), lambda g: (0, g)),
        ],
        out_specs=pl.BlockSpec((_TCHUNK, dim), lambda g: (g, 0)),
        out_shape=jax.ShapeDtypeStruct((vocab, dim), jnp.float32),
    )


@functools.lru_cache(maxsize=None)
def _make_gather(batch, vocab, dim):
    assert batch % (8 * _NW) == 0 and dim == 64
    b_per_w = batch // _NW
    mesh = plsc.VectorSubcoreMesh(core_axis_name="c", subcore_axis_name="s")

    @functools.partial(
        pl.kernel,
        mesh=mesh,
        out_type=jax.ShapeDtypeStruct((batch, dim), jnp.float32),
        scratch_types=[
            pltpu.VMEM((b_per_w,), jnp.int32),
            pltpu.VMEM((b_per_w, dim), jnp.float32),
            pltpu.SemaphoreType.DMA,
        ],
    )
    def gather(table_hbm, idx_hbm, out_hbm, idx_v, rows_v, sem):
        wid = lax.axis_index("s") * _NC + lax.axis_index("c")
        base = wid * b_per_w
        pltpu.sync_copy(idx_hbm.at[pl.ds(base, b_per_w)], idx_v)

        def group(g, _):
            i0 = g * _GRP
            idx_vec = idx_v[pl.ds(i0, _GRP)]
            copies = []
            for j in range(_GRP):
                r = idx_vec[j]
                copies.append(
                    pltpu.async_copy(
                        table_hbm.at[pl.ds(r, 1), :],
                        rows_v.at[pl.ds(i0 + j, 1), :],
                        sem,
                    )
                )
            for c in copies:
                c.wait()
            return ()

        lax.fori_loop(0, b_per_w // _GRP, group, (), unroll=False)
        pltpu.sync_copy(rows_v, out_hbm.at[pl.ds(base, b_per_w)])

    return gather


@jax.jit
def kernel(codes, emb_weight):
    batch, = codes.shape
    vocab, dim = emb_weight.shape
    main = (vocab // _TCHUNK) * _TCHUNK
    table_rm = _make_transpose(vocab, dim)(emb_weight.T)
    if main != vocab:
        table_rm = lax.dynamic_update_slice(
            table_rm, emb_weight[main:, :], (main, 0)
        )
    gather = _make_gather(batch, vocab, dim)
    return gather(table_rm, codes.astype(jnp.int32))
